# baseline (speedup 1.0000x reference)
import jax
import jax.numpy as jnp
from jax import lax
from jax.experimental import pallas as pl
from jax.experimental.pallas import tpu as pltpu
from jax.experimental.pallas import tpu_sc as plsc

_NC = 2
_NS = 16
_NW = _NC * _NS
_CHUNK_ROWS = 16


def _sc_copy_body(table_hbm, out_hbm, buf0, buf1, isem0, isem1, osem0, osem1):
    rows = out_hbm.shape[0]
    rows_per_worker = rows // _NW
    n_chunks = rows_per_worker // _CHUNK_ROWS
    wid = lax.axis_index("s") * _NC + lax.axis_index("c")
    base = wid * rows_per_worker
    bufs = (buf0, buf1)
    isems = (isem0, isem1)
    osems = (osem0, osem1)

    def in_copy(c):
        return pltpu.make_async_copy(
            table_hbm.at[pl.ds(base + c * _CHUNK_ROWS, _CHUNK_ROWS), :],
            bufs[c % 2], isems[c % 2])

    def out_copy(c):
        return pltpu.make_async_copy(
            bufs[c % 2],
            out_hbm.at[pl.ds(base + c * _CHUNK_ROWS, _CHUNK_ROWS), :],
            osems[c % 2])

    in_copy(0).start()
    for c in range(n_chunks):
        in_copy(c).wait()
        out_copy(c).start()
        if c + 1 < n_chunks:
            if c >= 1:
                out_copy(c - 1).wait()
            in_copy(c + 1).start()
    out_copy(n_chunks - 2).wait()
    out_copy(n_chunks - 1).wait()


def kernel(x, table):
    T = x.shape[1]
    _, d = table.shape
    mesh = plsc.VectorSubcoreMesh(core_axis_name="c", subcore_axis_name="s")
    f = pl.kernel(
        _sc_copy_body,
        mesh=mesh,
        out_type=jax.ShapeDtypeStruct((T, d), table.dtype),
        scratch_types=[
            pltpu.VMEM((_CHUNK_ROWS, d), table.dtype),
            pltpu.VMEM((_CHUNK_ROWS, d), table.dtype),
            pltpu.SemaphoreType.DMA,
            pltpu.SemaphoreType.DMA,
            pltpu.SemaphoreType.DMA,
            pltpu.SemaphoreType.DMA,
        ],
    )
    return f(table)
